# baseline (speedup 1.0000x reference)
import functools

import jax
import jax.numpy as jnp
from jax import lax
from jax.experimental import pallas as pl
from jax.experimental.pallas import tpu as pltpu
from jax.experimental.pallas import tpu_sc as plsc

NC = 2
NS = 16
NW = NC * NS
R = 12
NRB = 384
NROW = 24576


def _add_chunk(accs, buf):
    accs = list(accs)
    for row in range(R):
        ct = row % 3
        for wi in range(8):
            for j in range(8):
                a = ct * 8 + j
                accs[a] = accs[a] + buf[row, wi, pl.ds(j * 16, 16)]
    return tuple(accs)


def _sc_pool_body(x3, out, bufa, bufb, accv, sema, semb):
    wid = lax.axis_index("s") * NC + lax.axis_index("c")
    ch_per_b = NRB // R
    for bi in range(2):
        b = wid * 2 + bi
        b0 = b * NRB
        pltpu.make_async_copy(x3.at[pl.ds(b0, R)], bufa, sema).start()
        pltpu.make_async_copy(x3.at[pl.ds(b0 + R, R)], bufb, semb).start()
        zero = jnp.zeros((16,), jnp.float32)
        accs0 = (zero,) * 24

        def pair(i, accs):
            pltpu.make_async_copy(x3.at[pl.ds(b0, R)], bufa, sema).wait()
            accs = _add_chunk(accs, bufa)
            na = jnp.minimum(b0 + (2 * i + 2) * R, NROW - R)
            pltpu.make_async_copy(x3.at[pl.ds(na, R)], bufa, sema).start()
            pltpu.make_async_copy(x3.at[pl.ds(b0, R)], bufb, semb).wait()
            accs = _add_chunk(accs, bufb)
            nb = jnp.minimum(b0 + (2 * i + 3) * R, NROW - R)
            pltpu.make_async_copy(x3.at[pl.ds(nb, R)], bufb, semb).start()
            return accs

        accs = lax.fori_loop(0, ch_per_b // 2, pair, accs0)
        pltpu.make_async_copy(x3.at[pl.ds(b0, R)], bufa, sema).wait()
        pltpu.make_async_copy(x3.at[pl.ds(b0, R)], bufb, semb).wait()
        for a in range(24):
            ct, j = a // 8, a % 8
            accv[pl.ds(ct * 128 + j * 16, 16)] = accs[a]
        pltpu.sync_copy(accv, out.at[b])


def _sc_pool(x3):
    mesh = plsc.VectorSubcoreMesh(core_axis_name="c", subcore_axis_name="s")
    f = functools.partial(
        pl.kernel,
        mesh=mesh,
        out_type=jax.ShapeDtypeStruct((64, 384), jnp.float32),
        scratch_types=[
            pltpu.VMEM((R, 8, 128), jnp.float32),
            pltpu.VMEM((R, 8, 128), jnp.float32),
            pltpu.VMEM((384,), jnp.float32),
            pltpu.SemaphoreType.DMA,
            pltpu.SemaphoreType.DMA,
        ],
    )(_sc_pool_body)
    return f(x3)


def _tc_epilogue(p_ref, w_ref, o_ref):
    o_ref[...] = jax.lax.dot_general(
        p_ref[...], w_ref[...],
        dimension_numbers=(((1,), (1,)), ((), ())),
        preferred_element_type=jnp.float32,
    ) * (1.0 / 1024.0)


def kernel(x, W):
    B, C, H, Wsp = x.shape
    E = W.shape[0]
    xt = jnp.transpose(x, (0, 2, 3, 1))
    x6 = xt.reshape(B, H, Wsp // 8, 8, C // 128, 128)
    x3 = jnp.transpose(x6, (0, 1, 2, 4, 3, 5)).reshape(B * H * (Wsp // 8) * (C // 128), 8, 128)
    pooled = _sc_pool(x3)
    return pl.pallas_call(
        _tc_epilogue,
        grid=(1,),
        in_specs=[
            pl.BlockSpec((B, C), lambda i: (0, 0)),
            pl.BlockSpec((E, C), lambda i: (0, 0)),
        ],
        out_specs=pl.BlockSpec((B, E), lambda i: (0, 0)),
        out_shape=jax.ShapeDtypeStruct((B, E), jnp.float32),
    )(pooled, W)


# speedup vs baseline: 1.7684x; 1.7684x over previous
import functools

import jax
import jax.numpy as jnp
from jax import lax
from jax.experimental import pallas as pl
from jax.experimental.pallas import tpu as pltpu
from jax.experimental.pallas import tpu_sc as plsc

NC = 2
NS = 16
NW = NC * NS
R = 48
NRB = 384
CH = NRB // R


def _accum(accs, buf):
    def group(g, accs_t):
        accs_l = list(accs_t)
        r0 = g * 3
        for dr in range(3):
            for wi in range(8):
                for j in range(8):
                    a = dr * 8 + j
                    accs_l[a] = accs_l[a] + buf[r0 + dr, wi, pl.ds(j * 16, 16)]
        return tuple(accs_l)

    return lax.fori_loop(0, R // 3, group, accs)


def _sc_pool_body(x3, out, bufa, bufb, accv, sema, semb):
    wid = lax.axis_index("s") * NC + lax.axis_index("c")
    for bi in range(2):
        b = wid * 2 + bi
        b0 = b * NRB

        def startA(idx):
            pltpu.make_async_copy(x3.at[pl.ds(idx, R)], bufa, sema).start()

        def startB(idx):
            pltpu.make_async_copy(x3.at[pl.ds(idx, R)], bufb, semb).start()

        def waitA():
            pltpu.make_async_copy(x3.at[pl.ds(b0, R)], bufa, sema).wait()

        def waitB():
            pltpu.make_async_copy(x3.at[pl.ds(b0, R)], bufb, semb).wait()

        startA(b0)
        startB(b0 + R)
        zero = jnp.zeros((16,), jnp.float32)
        accs0 = (zero,) * 24

        def pair(i, accs):
            waitA()
            accs = _accum(accs, bufa)
            startA(b0 + (2 * i + 2) * R)
            waitB()
            accs = _accum(accs, bufb)
            startB(b0 + (2 * i + 3) * R)
            return accs

        accs = lax.fori_loop(0, CH // 2 - 1, pair, accs0)
        waitA()
        accs = _accum(accs, bufa)
        waitB()
        accs = _accum(accs, bufb)
        for a in range(24):
            ct, j = a // 8, a % 8
            accv[pl.ds(ct * 128 + j * 16, 16)] = accs[a]
        pltpu.sync_copy(accv, out.at[b])


def _sc_pool(x3):
    mesh = plsc.VectorSubcoreMesh(core_axis_name="c", subcore_axis_name="s")
    f = functools.partial(
        pl.kernel,
        mesh=mesh,
        out_type=jax.ShapeDtypeStruct((64, 384), jnp.float32),
        scratch_types=[
            pltpu.VMEM((R, 8, 128), jnp.float32),
            pltpu.VMEM((R, 8, 128), jnp.float32),
            pltpu.VMEM((384,), jnp.float32),
            pltpu.SemaphoreType.DMA,
            pltpu.SemaphoreType.DMA,
        ],
    )(_sc_pool_body)
    return f(x3)


def _tc_epilogue(p_ref, w_ref, o_ref):
    o_ref[...] = jax.lax.dot_general(
        p_ref[...], w_ref[...],
        dimension_numbers=(((1,), (1,)), ((), ())),
        preferred_element_type=jnp.float32,
    ) * (1.0 / 1024.0)


def kernel(x, W):
    B, C, H, Wsp = x.shape
    E = W.shape[0]
    xt = jnp.transpose(x, (0, 2, 3, 1))
    x6 = xt.reshape(B, H, Wsp // 8, 8, C // 128, 128)
    x3 = jnp.transpose(x6, (0, 1, 2, 4, 3, 5)).reshape(B * H * (Wsp // 8) * (C // 128), 8, 128)
    pooled = _sc_pool(x3)
    return pl.pallas_call(
        _tc_epilogue,
        grid=(1,),
        in_specs=[
            pl.BlockSpec((B, C), lambda i: (0, 0)),
            pl.BlockSpec((E, C), lambda i: (0, 0)),
        ],
        out_specs=pl.BlockSpec((B, E), lambda i: (0, 0)),
        out_shape=jax.ShapeDtypeStruct((B, E), jnp.float32),
    )(pooled, W)
